# baseline (speedup 1.0000x reference)
import functools
import numpy as np
import jax
import jax.numpy as jnp
from jax.experimental import pallas as pl
from jax.experimental.pallas import tpu as pltpu

IN_C, OUT_C, KSIZE, IMG = 3, 8, 5, 32
POOL_HW = 14
FC_OUT = 10
TB = 128
NK = 160
FLAT = IN_C * IMG * IMG
NF = 256


def _chan_weights_t(Wc):
    n = np.arange(NF)
    xpar = n // 128
    j = ((n % 128) // OUT_C) % 16
    o = n % OUT_C
    k = np.arange(NK)
    yloc = k // IMG
    xin = k % IMG
    kx = xin[:, None] - (2 * j + xpar)[None, :]
    valid = (kx >= 0) & (kx < KSIZE) & (j[None, :] < POOL_HW)
    ws = []
    for c in range(IN_C):
        src = ((o[None, :] * IN_C + c) * KSIZE + yloc[:, None]) * KSIZE \
            + np.clip(kx, 0, KSIZE - 1)
        wb = jnp.where(jnp.asarray(valid), Wc.reshape(-1)[jnp.asarray(src)], 0.0)
        ws.append(wb.astype(jnp.bfloat16))
    return jnp.stack(ws)


def _fc_weight_t(Wf):
    w4 = Wf.reshape(FC_OUT, OUT_C, POOL_HW, POOL_HW)
    w5 = jnp.transpose(w4, (2, 3, 1, 0)).reshape(POOL_HW, 112, FC_OUT)
    w5 = jnp.pad(w5, ((0, 0), (0, 16), (0, 6)))
    return w5.astype(jnp.bfloat16)


_DN = (((1,), (0,)), ((), ()))


def _net_kernel(x_ref, wt_ref, bcl_ref, wf_ref, bfl_ref, out_ref, xb_ref):
    xb_ref[...] = x_ref[0].reshape(TB, FLAT).astype(jnp.bfloat16)
    wts = [wt_ref[0], wt_ref[1], wt_ref[2]]
    bcl = bcl_ref[...]
    accs = [jnp.zeros(out_ref.shape, jnp.float32) for _ in range(2)]
    for i in range(POOL_HW):
        rs = []
        for ypar in range(2):
            y0 = (2 * i + ypar) * IMG
            r = None
            for c in range(IN_C):
                xs = xb_ref[:, c * IMG * IMG + y0:c * IMG * IMG + y0 + NK]
                d = jax.lax.dot_general(xs, wts[c], _DN,
                                        preferred_element_type=jnp.float32)
                r = d if r is None else r + d
            rs.append(r)
        ry = jnp.maximum(rs[0], rs[1])
        m = jnp.maximum(ry[:, :128], ry[:, 128:])
        a = jnp.maximum(m + bcl, 0.0).astype(jnp.bfloat16)
        accs[i % 2] = accs[i % 2] + jax.lax.dot_general(
            a, wf_ref[i], _DN, preferred_element_type=jnp.float32)
    out_ref[...] = accs[0] + accs[1] + bfl_ref[...]


@jax.jit
def _forward(x, Wc, bc, Wf, bf):
    B = x.shape[0]
    grid = pl.cdiv(B, TB)
    Bp = grid * TB
    if Bp != B:
        x = jnp.pad(x, ((0, Bp - B), (0, 0), (0, 0), (0, 0)))
    xr = x.reshape(grid, TB, IN_C, IMG, IMG)

    wt = _chan_weights_t(Wc)
    bcn = np.zeros((1, 128), np.float32)
    bcl = jnp.asarray(bcn) + jnp.tile(bc.astype(jnp.float32), 16).reshape(1, 128)
    wf_r = _fc_weight_t(Wf)
    bfl = jnp.pad(bf.astype(jnp.float32), (0, 6)).reshape(1, 16)

    flops = 2 * Bp * POOL_HW * (6 * NK * NF + 128 * 16)
    bytes_accessed = grid * TB * FLAT * 4 + 3 * NK * NF * 2 + Bp * 16 * 4

    out = pl.pallas_call(
        _net_kernel,
        out_shape=jax.ShapeDtypeStruct((Bp, 16), jnp.float32),
        grid=(grid,),
        in_specs=[
            pl.BlockSpec((1, TB, IN_C, IMG, IMG), lambda b: (b, 0, 0, 0, 0)),
            pl.BlockSpec((IN_C, NK, NF), lambda b: (0, 0, 0)),
            pl.BlockSpec((1, 128), lambda b: (0, 0)),
            pl.BlockSpec((POOL_HW, 128, 16), lambda b: (0, 0, 0)),
            pl.BlockSpec((1, 16), lambda b: (0, 0)),
        ],
        out_specs=pl.BlockSpec((TB, 16), lambda b: (b, 0)),
        scratch_shapes=[pltpu.VMEM((TB, FLAT), jnp.bfloat16)],
        compiler_params=pltpu.CompilerParams(
            dimension_semantics=("parallel",),
        ),
        cost_estimate=pl.CostEstimate(flops=int(flops), transcendentals=0,
                                      bytes_accessed=int(bytes_accessed)),
    )(xr, wt, bcl, wf_r, bfl)
    return out[:B, :FC_OUT]


def kernel(x, Wc, bc, Wf, bf):
    return _forward(x, Wc, bc, Wf, bf)


# speedup vs baseline: 1.9188x; 1.9188x over previous
import functools
import numpy as np
import jax
import jax.numpy as jnp
from jax.experimental import pallas as pl
from jax.experimental.pallas import tpu as pltpu

IN_C = 3
OUT_C = 8
KSIZE = 5
IMG = 32
POOL_HW = 14
FC_OUT = 10
F_PAD = 16
TB = 256

GW = 16
NW = 2
GROW = IN_C * GW
WK = 5 * GROW
WH = IMG * GROW
WM = 128
JL = 8


def _window_weights(Wc):
    m = np.arange(WM)
    xpar = m // 64
    jl = (m // OUT_C) % JL
    o = m % OUT_C
    k = np.arange(WK)
    yloc = k // GROW
    c = (k // GW) % IN_C
    dx = k % GW
    ws = []
    for w in range(NW):
        kx = (4 * w + dx)[None, :] - (2 * jl + xpar)[:, None]
        valid = (kx >= 0) & (kx < KSIZE)
        src = ((o[:, None] * IN_C + c[None, :]) * KSIZE + yloc[None, :]) * KSIZE \
            + np.clip(kx, 0, KSIZE - 1)
        wb = jnp.where(jnp.asarray(valid), Wc.reshape(-1)[jnp.asarray(src)], 0.0)
        ws.append(wb.astype(jnp.bfloat16))
    return jnp.stack(ws)


def _fc_weight(Wf):
    w4 = Wf.reshape(FC_OUT, OUT_C, POOL_HW, POOL_HW)
    w4 = jnp.transpose(w4, (2, 0, 3, 1)).reshape(POOL_HW, FC_OUT, 112)
    w4 = jnp.pad(w4, ((0, 0), (0, F_PAD - FC_OUT), (0, 0)))
    return w4.astype(jnp.bfloat16)


def _net_kernel(x_ref, w_ref, bct_ref, wf_ref, bf_ref, out_ref):
    w0 = w_ref[0]
    w1 = w_ref[1]
    bcA = bct_ref[0:48]
    bcB = bct_ref[48:64]
    accs = [jnp.zeros(out_ref.shape, jnp.float32) for _ in range(2)]
    for i in range(POOL_HW):
        r = []
        for w in range(NW):
            wm = w0 if w == 0 else w1
            for ypar in range(2):
                base = w * WH + 96 * i + 48 * ypar
                r.append(jnp.dot(wm, x_ref[0, base:base + WK, :],
                                 preferred_element_type=jnp.float32))
        r00, r01, r10, r11 = r
        mA = jnp.maximum(jnp.maximum(r00[0:48], r00[64:112]),
                         jnp.maximum(r01[0:48], r01[64:112]))
        mC = jnp.maximum(jnp.maximum(r10[16:64], r10[80:128]),
                         jnp.maximum(r11[16:64], r11[80:128]))
        s0 = jnp.maximum(r00[48:64] + r10[0:16], r00[112:128] + r10[64:80])
        s1 = jnp.maximum(r01[48:64] + r11[0:16], r01[112:128] + r11[64:80])
        mB = jnp.maximum(s0, s1)
        a = jnp.concatenate([
            jnp.maximum(mA + bcA, 0.0).astype(jnp.bfloat16),
            jnp.maximum(mB + bcB, 0.0).astype(jnp.bfloat16),
            jnp.maximum(mC + bcA, 0.0).astype(jnp.bfloat16),
        ], axis=0)
        accs[i % 2] = accs[i % 2] + jnp.dot(
            wf_ref[i], a, preferred_element_type=jnp.float32)
    out_ref[...] = accs[0] + accs[1] + bf_ref[...]


@jax.jit
def _forward(x, Wc, bc, Wf, bf):
    B = x.shape[0]
    grid = pl.cdiv(B, TB)
    Bp = grid * TB

    xb = x.astype(jnp.bfloat16)
    if Bp != B:
        xb = jnp.pad(xb, ((0, Bp - B), (0, 0), (0, 0), (0, 0)))
    xt = xb.reshape(grid, TB, IN_C, IMG, NW, GW)
    xt = jnp.transpose(xt, (0, 4, 3, 2, 5, 1)).reshape(grid, NW * WH, TB)

    wcw = _window_weights(Wc)
    bcf = bc.astype(jnp.float32)
    bct = jnp.concatenate([jnp.tile(bcf, 6), jnp.tile(bcf, 2)]).reshape(64, 1)
    wf_r = _fc_weight(Wf)
    bf_col = jnp.pad(bf.astype(jnp.float32), (0, F_PAD - FC_OUT)).reshape(F_PAD, 1)

    flops = 2 * Bp * POOL_HW * (4 * WM * WK + F_PAD * 112)
    bytes_accessed = (grid * NW * WH * TB * 2 + NW * WM * WK * 2
                      + POOL_HW * F_PAD * 112 * 2 + 64 * 4 + F_PAD * 4
                      + F_PAD * Bp * 4)

    out = pl.pallas_call(
        _net_kernel,
        out_shape=jax.ShapeDtypeStruct((F_PAD, Bp), jnp.float32),
        grid=(grid,),
        in_specs=[
            pl.BlockSpec((1, NW * WH, TB), lambda b: (b, 0, 0)),
            pl.BlockSpec((NW, WM, WK), lambda b: (0, 0, 0)),
            pl.BlockSpec((64, 1), lambda b: (0, 0)),
            pl.BlockSpec((POOL_HW, F_PAD, 112), lambda b: (0, 0, 0)),
            pl.BlockSpec((F_PAD, 1), lambda b: (0, 0)),
        ],
        out_specs=pl.BlockSpec((F_PAD, TB), lambda b: (0, b)),
        compiler_params=pltpu.CompilerParams(
            dimension_semantics=("parallel",),
        ),
        cost_estimate=pl.CostEstimate(flops=int(flops), transcendentals=0,
                                      bytes_accessed=int(bytes_accessed)),
    )(xt, wcw, bct, wf_r, bf_col)
    return jnp.transpose(out[:FC_OUT, :B])


def kernel(x, Wc, bc, Wf, bf):
    return _forward(x, Wc, bc, Wf, bf)
